# baseline (speedup 1.0000x reference)
import functools

import jax
import jax.numpy as jnp
from jax.experimental import pallas as pl
from jax.experimental.pallas import tpu as pltpu

N_ROIS = 1000
C = 64
POOL = 7
RATIO = 2
D_HEAD = 1024
N_CLASSES = 21
THRESH = 0.05
SCALE = 1.0 / 16.0
NPAD = 1024
K_POST = 100
MB = 256


def _roi_align_host(feat, rois):
    H, W, _ = feat.shape
    x1 = rois[:, 1] * SCALE
    y1 = rois[:, 2] * SCALE
    x2 = rois[:, 3] * SCALE
    y2 = rois[:, 4] * SCALE
    rw = jnp.maximum(x2 - x1, 1.0)
    rh = jnp.maximum(y2 - y1, 1.0)
    bw = rw / POOL
    bh = rh / POOL
    idx = jnp.arange(POOL, dtype=jnp.float32)
    s = (jnp.arange(RATIO, dtype=jnp.float32) + 0.5) / RATIO
    off = idx[:, None] + s[None, :]
    ys = y1[:, None, None] + bh[:, None, None] * off[None]
    xs = x1[:, None, None] + bw[:, None, None] * off[None]
    Y = ys[:, :, :, None, None]
    X = xs[:, None, None, :, :]
    Yc = jnp.clip(Y, 0.0, H - 1.0)
    Xc = jnp.clip(X, 0.0, W - 1.0)
    y0 = jnp.floor(Yc)
    x0 = jnp.floor(Xc)
    wy = Yc - y0
    wx = Xc - x0
    y0i = y0.astype(jnp.int32)
    x0i = x0.astype(jnp.int32)
    y1i = jnp.clip(y0i + 1, 0, H - 1)
    x1i = jnp.clip(x0i + 1, 0, W - 1)
    v00 = feat[y0i, x0i]
    v01 = feat[y0i, x1i]
    v10 = feat[y1i, x0i]
    v11 = feat[y1i, x1i]
    wy = wy[..., None]
    wx = wx[..., None]
    val = v00 * (1 - wy) * (1 - wx) + v01 * (1 - wy) * wx + v10 * wy * (1 - wx) + v11 * wy * wx
    pooled = val.mean(axis=(2, 4))
    return pooled


def _head_body(flat_ref, wh_ref, bh_ref, wc_ref, bc_ref,
               head_out, prob_out, sel_out):
    flat = flat_ref[...]
    head = jnp.dot(flat, wh_ref[...], preferred_element_type=jnp.float32)
    head = jnp.maximum(head + bh_ref[...], 0.0)
    head_out[...] = head
    cls_score = jnp.dot(head, wc_ref[...], preferred_element_type=jnp.float32)
    cls_score = cls_score + bc_ref[...]
    xm = jnp.max(cls_score, axis=1, keepdims=True)
    e = jnp.exp(cls_score - xm)
    ssum = jnp.sum(e, axis=1, keepdims=True)
    prob = e / ssum
    prob_out[...] = prob
    maxp = jnp.max(prob, axis=1, keepdims=True)
    keep = (prob[:, 0:1] < maxp) & (maxp > THRESH)
    sel_out[...] = jnp.where(keep, maxp, -1.0)


_head_call = pl.pallas_call(
    _head_body,
    grid=(NPAD // MB,),
    in_specs=[
        pl.BlockSpec((MB, POOL * POOL * C), lambda i: (i, 0)),
        pl.BlockSpec((POOL * POOL * C, D_HEAD), lambda i: (0, 0)),
        pl.BlockSpec((1, D_HEAD), lambda i: (0, 0)),
        pl.BlockSpec((D_HEAD, N_CLASSES), lambda i: (0, 0)),
        pl.BlockSpec((1, N_CLASSES), lambda i: (0, 0)),
    ],
    out_specs=(
        pl.BlockSpec((MB, D_HEAD), lambda i: (i, 0)),
        pl.BlockSpec((MB, N_CLASSES), lambda i: (i, 0)),
        pl.BlockSpec((MB, 1), lambda i: (i, 0)),
    ),
    out_shape=(
        jax.ShapeDtypeStruct((NPAD, D_HEAD), jnp.float32),
        jax.ShapeDtypeStruct((NPAD, N_CLASSES), jnp.float32),
        jax.ShapeDtypeStruct((NPAD, 1), jnp.float32),
    ),
)


def _topk_body(sel_ref, rois_ref, head_ref, prob_ref,
               rois_out, feat_out, prob_out):
    s = jnp.transpose(sel_ref[...])
    lane = jax.lax.broadcasted_iota(jnp.int32, (1, NPAD), 1)

    def body(i, s):
        m = jnp.max(s)
        idx = jnp.min(jnp.where(s == m, lane, NPAD))
        rois_out[pl.ds(i, 1), :] = rois_ref[pl.ds(idx, 1), :]
        feat_out[pl.ds(i, 1), :] = head_ref[pl.ds(idx, 1), :]
        prob_out[pl.ds(i, 1), :] = prob_ref[pl.ds(idx, 1), :]
        return jnp.where(lane == idx, -2.0, s)

    jax.lax.fori_loop(0, K_POST, body, s)


_topk_call = pl.pallas_call(
    _topk_body,
    out_shape=(
        jax.ShapeDtypeStruct((K_POST, 5), jnp.float32),
        jax.ShapeDtypeStruct((K_POST, D_HEAD), jnp.float32),
        jax.ShapeDtypeStruct((K_POST, N_CLASSES), jnp.float32),
    ),
)


def kernel(base_feat, rois, W_head, b_head, W_cls, b_cls):
    feat = base_feat[0]
    pooled = _roi_align_host(feat, rois)
    flat = pooled.reshape((rois.shape[0], -1))
    flat_pad = jnp.zeros((NPAD, POOL * POOL * C), jnp.float32).at[:N_ROIS].set(flat)
    rois_pad = jnp.zeros((NPAD, 5), jnp.float32).at[:N_ROIS].set(rois)
    head, prob, sel = _head_call(flat_pad, W_head, b_head.reshape(1, -1),
                                 W_cls, b_cls.reshape(1, -1))
    post_rois, post_feat, post_prob = _topk_call(sel, rois_pad, head, prob)
    return post_rois[None], post_feat, post_prob[None]


# speedup vs baseline: 22.4154x; 22.4154x over previous
import dataclasses
import functools

import jax
import jax.numpy as jnp
from jax import lax
from jax.experimental import pallas as pl
from jax.experimental.pallas import tpu as pltpu
from jax.experimental.pallas import tpu_sc as plsc

N_ROIS = 1000
C = 64
POOL = 7
RATIO = 2
D_HEAD = 1024
N_CLASSES = 21
THRESH = 0.05
SCALE = 1.0 / 16.0
NPAD = 1024
K_POST = 100
MB = 256


def _roi_align_host(feat, rois):
    H, W, _ = feat.shape
    x1 = rois[:, 1] * SCALE
    y1 = rois[:, 2] * SCALE
    x2 = rois[:, 3] * SCALE
    y2 = rois[:, 4] * SCALE
    rw = jnp.maximum(x2 - x1, 1.0)
    rh = jnp.maximum(y2 - y1, 1.0)
    bw = rw / POOL
    bh = rh / POOL
    idx = jnp.arange(POOL, dtype=jnp.float32)
    s = (jnp.arange(RATIO, dtype=jnp.float32) + 0.5) / RATIO
    off = idx[:, None] + s[None, :]
    ys = y1[:, None, None] + bh[:, None, None] * off[None]
    xs = x1[:, None, None] + bw[:, None, None] * off[None]
    Y = ys[:, :, :, None, None]
    X = xs[:, None, None, :, :]
    Yc = jnp.clip(Y, 0.0, H - 1.0)
    Xc = jnp.clip(X, 0.0, W - 1.0)
    y0 = jnp.floor(Yc)
    x0 = jnp.floor(Xc)
    wy = Yc - y0
    wx = Xc - x0
    y0i = y0.astype(jnp.int32)
    x0i = x0.astype(jnp.int32)
    y1i = jnp.clip(y0i + 1, 0, H - 1)
    x1i = jnp.clip(x0i + 1, 0, W - 1)
    v00 = feat[y0i, x0i]
    v01 = feat[y0i, x1i]
    v10 = feat[y1i, x0i]
    v11 = feat[y1i, x1i]
    wy = wy[..., None]
    wx = wx[..., None]
    val = v00 * (1 - wy) * (1 - wx) + v01 * (1 - wy) * wx + v10 * wy * (1 - wx) + v11 * wy * wx
    pooled = val.mean(axis=(2, 4))
    return pooled


H_FEAT = 50
W_FEAT = 75
ROI_GROUPS = 8
CH_GROUPS = 4
ROIS_PER_TILE = NPAD // ROI_GROUPS
CH_PER_TILE = C // CH_GROUPS

_SC_MESH = plsc.VectorSubcoreMesh(core_axis_name="c", subcore_axis_name="s")

_SC_PARAMS = pltpu.CompilerParams()
if "needs_layout_passes" in pltpu.CompilerParams.__dataclass_fields__:
    _SC_PARAMS = dataclasses.replace(_SC_PARAMS, needs_layout_passes=False)


def _sc_roi_align_body(feat_hbm, rois_hbm, out_hbm,
                       feat_v, rois_v, y0_b, y1_b, wy_b, x0_b, x1_b, wx_b,
                       pooled_v):
    cidx = lax.axis_index("c")
    sidx = lax.axis_index("s")
    wid = sidx * 2 + cidx
    cg = lax.rem(wid, CH_GROUPS)
    rg = lax.div(wid, CH_GROUPS)
    c0 = cg * CH_PER_TILE
    r0 = rg * ROIS_PER_TILE
    pltpu.sync_copy(feat_hbm.at[pl.ds(c0, CH_PER_TILE), :], feat_v)
    pltpu.sync_copy(rois_hbm.at[pl.ds(r0, ROIS_PER_TILE), :], rois_v)

    lane = lax.iota(jnp.int32, 16)
    off = lane.astype(jnp.float32) * 0.5 + 0.25

    @pl.loop(0, ROIS_PER_TILE)
    def _roi_loop(ri):
        riv = jnp.full((16,), ri, jnp.int32)

        def splat_col(col):
            return plsc.load_gather(rois_v, [riv, jnp.full((16,), col, jnp.int32)])

        x1 = splat_col(1) * SCALE
        y1 = splat_col(2) * SCALE
        x2 = splat_col(3) * SCALE
        y2 = splat_col(4) * SCALE
        rw = jnp.maximum(x2 - x1, 1.0)
        rh = jnp.maximum(y2 - y1, 1.0)
        bw = rw / float(POOL)
        bh = rh / float(POOL)
        ys = y1 + bh * off
        xs = x1 + bw * off
        yc = jnp.clip(ys, 0.0, float(H_FEAT - 1))
        xc = jnp.clip(xs, 0.0, float(W_FEAT - 1))
        y0i = yc.astype(jnp.int32)
        x0i = xc.astype(jnp.int32)
        wy = yc - y0i.astype(jnp.float32)
        wx = xc - x0i.astype(jnp.float32)
        y0_b[...] = y0i
        y1_b[...] = jnp.minimum(y0i + 1, H_FEAT - 1)
        wy_b[...] = wy
        x0_b[...] = x0i
        x1_b[...] = jnp.minimum(x0i + 1, W_FEAT - 1)
        wx_b[...] = wx

        @pl.loop(0, 4)
        def _bin_loop(v):
            b = jnp.minimum(lane + v * 16, POOL * POOL - 1)
            by = lax.div(b, POOL)
            bx = lax.rem(b, POOL)
            acc = [None] * CH_PER_TILE
            for qy, qx in ((0, 0), (1, 0), (0, 1), (1, 1)):
                i_idx = by * 2 + qy
                j_idx = bx * 2 + qx
                yy0 = plsc.load_gather(y0_b, [i_idx])
                yy1 = plsc.load_gather(y1_b, [i_idx])
                wyv = plsc.load_gather(wy_b, [i_idx])
                xx0 = plsc.load_gather(x0_b, [j_idx])
                xx1 = plsc.load_gather(x1_b, [j_idx])
                wxv = plsc.load_gather(wx_b, [j_idx])
                onemwy = 1.0 - wyv
                onemwx = 1.0 - wxv
                r00 = yy0 * 128 + xx0
                r01 = yy0 * 128 + xx1
                r10 = yy1 * 128 + xx0
                r11 = yy1 * 128 + xx1
                for cc in range(CH_PER_TILE):
                    ccv = jnp.full((16,), cc, jnp.int32)
                    v00 = plsc.load_gather(feat_v, [ccv, r00])
                    v01 = plsc.load_gather(feat_v, [ccv, r01])
                    v10 = plsc.load_gather(feat_v, [ccv, r10])
                    v11 = plsc.load_gather(feat_v, [ccv, r11])
                    t = (((v00 * onemwy) * onemwx + (v01 * onemwy) * wxv)
                         + (v10 * wyv) * onemwx) + (v11 * wyv) * wxv
                    acc[cc] = t if qy == 0 and qx == 0 else acc[cc] + t
            for cc in range(CH_PER_TILE):
                plsc.store_scatter(pooled_v, [b, jnp.full((16,), cc, jnp.int32)],
                                   acc[cc] * 0.25)

        pltpu.sync_copy(pooled_v.at[pl.ds(0, POOL * POOL), :],
                        out_hbm.at[cg, r0 + ri, :, :])


_sc_roi_align = functools.partial(
    pl.kernel,
    mesh=_SC_MESH,
    compiler_params=_SC_PARAMS,
    out_type=jax.ShapeDtypeStruct((CH_GROUPS, NPAD, POOL * POOL, CH_PER_TILE),
                                  jnp.float32),
    scratch_types=[
        pltpu.VMEM((CH_PER_TILE, H_FEAT * 128), jnp.float32),
        pltpu.VMEM((ROIS_PER_TILE, 5), jnp.float32),
        pltpu.VMEM((16,), jnp.int32),
        pltpu.VMEM((16,), jnp.int32),
        pltpu.VMEM((16,), jnp.float32),
        pltpu.VMEM((16,), jnp.int32),
        pltpu.VMEM((16,), jnp.int32),
        pltpu.VMEM((16,), jnp.float32),
        pltpu.VMEM((POOL * POOL + 15, CH_PER_TILE), jnp.float32),
    ],
)(_sc_roi_align_body)


def _head_body(flat_ref, wh_ref, bh_ref, wc_ref, bc_ref,
               head_out, prob_out, sel_out):
    flat = flat_ref[...]
    head = jnp.dot(flat, wh_ref[...], preferred_element_type=jnp.float32)
    head = jnp.maximum(head + bh_ref[...], 0.0)
    head_out[...] = head
    cls_score = jnp.dot(head, wc_ref[...], preferred_element_type=jnp.float32)
    cls_score = cls_score + bc_ref[...]
    xm = jnp.max(cls_score, axis=1, keepdims=True)
    e = jnp.exp(cls_score - xm)
    ssum = jnp.sum(e, axis=1, keepdims=True)
    prob = e / ssum
    prob_out[...] = prob
    maxp = jnp.max(prob, axis=1, keepdims=True)
    keep = (prob[:, 0:1] < maxp) & (maxp > THRESH)
    sel_out[...] = jnp.where(keep, maxp, -1.0)


_head_call = pl.pallas_call(
    _head_body,
    grid=(NPAD // MB,),
    in_specs=[
        pl.BlockSpec((MB, POOL * POOL * C), lambda i: (i, 0)),
        pl.BlockSpec((POOL * POOL * C, D_HEAD), lambda i: (0, 0)),
        pl.BlockSpec((1, D_HEAD), lambda i: (0, 0)),
        pl.BlockSpec((D_HEAD, N_CLASSES), lambda i: (0, 0)),
        pl.BlockSpec((1, N_CLASSES), lambda i: (0, 0)),
    ],
    out_specs=(
        pl.BlockSpec((MB, D_HEAD), lambda i: (i, 0)),
        pl.BlockSpec((MB, N_CLASSES), lambda i: (i, 0)),
        pl.BlockSpec((MB, 1), lambda i: (i, 0)),
    ),
    out_shape=(
        jax.ShapeDtypeStruct((NPAD, D_HEAD), jnp.float32),
        jax.ShapeDtypeStruct((NPAD, N_CLASSES), jnp.float32),
        jax.ShapeDtypeStruct((NPAD, 1), jnp.float32),
    ),
)


def _topk_body(sel_ref, rois_ref, head_ref, prob_ref,
               rois_out, feat_out, prob_out):
    s = jnp.transpose(sel_ref[...])
    lane = jax.lax.broadcasted_iota(jnp.int32, (1, NPAD), 1)

    def body(i, s):
        m = jnp.max(s)
        idx = jnp.min(jnp.where(s == m, lane, NPAD))
        rois_out[pl.ds(i, 1), :] = rois_ref[pl.ds(idx, 1), :]
        feat_out[pl.ds(i, 1), :] = head_ref[pl.ds(idx, 1), :]
        prob_out[pl.ds(i, 1), :] = prob_ref[pl.ds(idx, 1), :]
        return jnp.where(lane == idx, -2.0, s)

    jax.lax.fori_loop(0, K_POST, body, s)


_topk_call = pl.pallas_call(
    _topk_body,
    out_shape=(
        jax.ShapeDtypeStruct((K_POST, 5), jnp.float32),
        jax.ShapeDtypeStruct((K_POST, D_HEAD), jnp.float32),
        jax.ShapeDtypeStruct((K_POST, N_CLASSES), jnp.float32),
    ),
)


def kernel(base_feat, rois, W_head, b_head, W_cls, b_cls):
    rois_pad = jnp.zeros((NPAD, 5), jnp.float32).at[:N_ROIS].set(rois)
    feat_t = jnp.transpose(base_feat[0], (2, 0, 1))
    feat_r = jnp.pad(feat_t, ((0, 0), (0, 0), (0, 128 - W_FEAT))).reshape(C, -1)
    pooled4 = _sc_roi_align(feat_r, rois_pad)
    flat_pad = pooled4.transpose(1, 2, 0, 3).reshape(NPAD, POOL * POOL * C)
    row = jax.lax.broadcasted_iota(jnp.int32, (NPAD, 1), 0)
    flat_pad = jnp.where(row < N_ROIS, flat_pad, 0.0)
    head, prob, sel = _head_call(flat_pad, W_head, b_head.reshape(1, -1),
                                 W_cls, b_cls.reshape(1, -1))
    post_rois, post_feat, post_prob = _topk_call(sel, rois_pad, head, prob)
    return post_rois[None], post_feat, post_prob[None]
